# baseline (speedup 1.0000x reference)
import functools

import jax
import jax.numpy as jnp
from jax import lax
from jax.experimental import pallas as pl
from jax.experimental.pallas import tpu as pltpu
from jax.experimental.pallas import tpu_sc as plsc

_NUM_CLASSES = 1000000
_OUT_DIM = 64
_BATCH = 16384
_L = 16
_NC = 2
_NS = 16
_NW = _NC * _NS
_B_PER_W = _BATCH // _NW

_mesh = plsc.VectorSubcoreMesh(core_axis_name="c", subcore_axis_name="s")


@functools.partial(
    pl.kernel,
    mesh=_mesh,
    out_type=jax.ShapeDtypeStruct((_BATCH, _OUT_DIM), jnp.float32),
    scratch_types=[
        pltpu.VMEM((_B_PER_W,), jnp.int32),
        pltpu.VMEM((_B_PER_W,), jnp.int32),
        pltpu.VMEM((_L,), jnp.int32),
        pltpu.VMEM((_B_PER_W, _OUT_DIM), jnp.float32),
        pltpu.SemaphoreType.DMA,
    ],
    compiler_params=pltpu.CompilerParams(use_tc_tiling_on_sc=False),
)
def _embed(labels_hbm, train_hbm, drop_hbm, table_hbm, out_hbm,
           idx_v, drop_v, train_v, rows_v, sem):
    wid = lax.axis_index("s") * _NC + lax.axis_index("c")
    base = wid * _B_PER_W
    pltpu.sync_copy(labels_hbm.at[pl.ds(base, _B_PER_W)], idx_v)
    pltpu.sync_copy(drop_hbm.at[pl.ds(base, _B_PER_W)], drop_v)
    pltpu.sync_copy(train_hbm, train_v)
    trn = train_v[...]
    null_row = jnp.full((_L,), _NUM_CLASSES, dtype=jnp.int32)
    for i in range(_B_PER_W // _L):
        sl = pl.ds(i * _L, _L)
        lab = idx_v[sl]
        drp = drop_v[sl]
        idx_v[sl] = jnp.where((trn != 0) & (drp != 0), null_row, lab)
    pltpu.async_copy(table_hbm.at[idx_v], rows_v, sem).wait()
    pltpu.sync_copy(rows_v, out_hbm.at[pl.ds(base, _B_PER_W)])


def kernel(labels, train, force_drop_ids, table):
    labels32 = labels.astype(jnp.int32)
    drop32 = force_drop_ids.astype(jnp.int32)
    train_vec = jnp.full((_L,), jnp.asarray(train, dtype=jnp.int32))
    return _embed(labels32, train_vec, drop32, table)


# speedup vs baseline: 1.1474x; 1.1474x over previous
import functools

import jax
import jax.numpy as jnp
from jax import lax
from jax.experimental import pallas as pl
from jax.experimental.pallas import tpu as pltpu
from jax.experimental.pallas import tpu_sc as plsc

_NUM_CLASSES = 1000000
_OUT_DIM = 64
_BATCH = 16384
_L = 16
_NC = 2
_NS = 16
_NW = _NC * _NS
_B_PER_W = _BATCH // _NW
_NG = _B_PER_W // _L

_mesh = plsc.VectorSubcoreMesh(core_axis_name="c", subcore_axis_name="s")


@functools.partial(
    pl.kernel,
    mesh=_mesh,
    out_type=jax.ShapeDtypeStruct((_BATCH, _OUT_DIM), jnp.float32),
    scratch_types=[
        pltpu.VMEM((_B_PER_W,), jnp.int32),
        pltpu.VMEM((_B_PER_W,), jnp.int32),
        pltpu.VMEM((_L,), jnp.int32),
        pltpu.VMEM((_B_PER_W, _OUT_DIM), jnp.float32),
        pltpu.SemaphoreType.DMA,
        pltpu.SemaphoreType.DMA,
    ],
)
def _embed(labels_hbm, train_hbm, drop_hbm, table_hbm, out_hbm,
           idx_v, drop_v, train_v, rows_v, sem, sem2):
    wid = lax.axis_index("s") * _NC + lax.axis_index("c")
    base = wid * _B_PER_W
    pltpu.sync_copy(labels_hbm.at[pl.ds(base, _B_PER_W)], idx_v)
    pltpu.sync_copy(drop_hbm.at[pl.ds(base, _B_PER_W)], drop_v)
    pltpu.sync_copy(train_hbm, train_v)
    trn = train_v[...]
    null_row = jnp.full((_L,), _NUM_CLASSES, dtype=jnp.int32)
    for i in range(_NG):
        sl = pl.ds(i * _L, _L)
        idx_v[sl] = jnp.where((trn != 0) & (drop_v[sl] != 0),
                              null_row, idx_v[sl])

    @pl.loop(0, _NG)
    def _(g):
        lab = idx_v[pl.ds(g * _L, _L)]
        for k in range(_L):
            pltpu.async_copy(
                table_hbm.at[pl.ds(lab[k], 1), :],
                rows_v.at[pl.ds(g * _L + k, 1), :], sem)

    pltpu.make_async_copy(
        table_hbm.at[pl.ds(0, _B_PER_W), :], rows_v, sem).wait()
    pltpu.async_copy(rows_v, out_hbm.at[pl.ds(base, _B_PER_W)], sem2).wait()


def kernel(labels, train, force_drop_ids, table):
    labels32 = labels.astype(jnp.int32)
    drop32 = force_drop_ids.astype(jnp.int32)
    train_vec = jnp.full((_L,), jnp.asarray(train, dtype=jnp.int32))
    return _embed(labels32, train_vec, drop32, table)
